# baseline (speedup 1.0000x reference)
import functools

import jax
import jax.numpy as jnp
from jax import lax
from jax.experimental import pallas as pl
from jax.experimental.pallas import tpu as pltpu
from jax.experimental.pallas import tpu_sc as plsc

B = 16384
EMB = 64
NC = 2
NS = 16
NW = NC * NS
BPW = B // NW
CH = 128
NCH = BPW // CH
NBLK = BPW // 16


def _mf_body(u_id, i_id, uemb, ubias, iemb, ibias, mean, out,
             idx_u, idx_i, idxq_u, idxq_i, U, I, Ub, Ib, outv, meanv, sem):
    wid = lax.axis_index("s") * NC + lax.axis_index("c")
    base = wid * BPW

    for c in range(NCH):
        pltpu.sync_copy(u_id.at[pl.ds(base + c * CH, CH)], idx_u.at[c])
        pltpu.sync_copy(i_id.at[pl.ds(base + c * CH, CH)], idx_i.at[c])
    pltpu.sync_copy(mean, meanv.at[pl.ds(0, 1)])

    for c in range(NCH):
        for j in range(CH // 16):
            sl = pl.ds(j * 16, 16)
            idxq_u[c, sl] = lax.shift_right_logical(idx_u[c, sl], 4)
            idxq_i[c, sl] = lax.shift_right_logical(idx_i[c, sl], 4)

    cps = []
    for c in range(NCH):
        sl = pl.ds(c * CH, CH)
        cps.append(pltpu.async_copy(uemb.at[idx_u.at[c]], U.at[sl], sem))
        cps.append(pltpu.async_copy(iemb.at[idx_i.at[c]], I.at[sl], sem))
        cps.append(pltpu.async_copy(ubias.at[idxq_u.at[c]], Ub.at[sl], sem))
        cps.append(pltpu.async_copy(ibias.at[idxq_i.at[c]], Ib.at[sl], sem))
    for cp in cps:
        cp.wait()

    mval = meanv[...][0]
    iota = lax.iota(jnp.int32, 16)
    fifteen = jnp.full((16,), 15, jnp.int32)

    def blk(b, carry):
        rows = b * 16 + iota
        acc = jnp.zeros((16,), jnp.float32)
        for d in range(EMB):
            cols = jnp.full((16,), d, jnp.int32)
            u = plsc.load_gather(U, [rows, cols])
            v = plsc.load_gather(I, [rows, cols])
            acc = acc + u * v
        c = b // 8
        sl = pl.ds((b % 8) * 16, 16)
        uvals = idx_u[c, sl]
        ivals = idx_i[c, sl]
        ub = plsc.load_gather(Ub, [rows, lax.bitwise_and(uvals, fifteen)])
        ib = plsc.load_gather(Ib, [rows, lax.bitwise_and(ivals, fifteen)])
        outv[pl.ds(b * 16, 16)] = acc + ub + ib + mval
        return carry

    lax.fori_loop(0, NBLK, blk, 0)

    pltpu.sync_copy(outv, out.at[pl.ds(base, BPW)])


@jax.jit
def kernel(u_id, i_id, user_emb_w, user_bias_w, item_emb_w, item_bias_w, mean):
    mesh = plsc.VectorSubcoreMesh(core_axis_name="c", subcore_axis_name="s")
    f = functools.partial(
        pl.kernel,
        out_type=jax.ShapeDtypeStruct((B,), jnp.float32),
        mesh=mesh,
        scratch_types=[
            pltpu.VMEM((NCH, CH), jnp.int32),
            pltpu.VMEM((NCH, CH), jnp.int32),
            pltpu.VMEM((NCH, CH), jnp.int32),
            pltpu.VMEM((NCH, CH), jnp.int32),
            pltpu.VMEM((BPW, EMB), jnp.float32),
            pltpu.VMEM((BPW, EMB), jnp.float32),
            pltpu.VMEM((BPW, 16), jnp.float32),
            pltpu.VMEM((BPW, 16), jnp.float32),
            pltpu.VMEM((BPW,), jnp.float32),
            pltpu.VMEM((16,), jnp.float32),
            pltpu.SemaphoreType.DMA,
        ],
        compiler_params=pltpu.CompilerParams(
            needs_layout_passes=False, use_tc_tiling_on_sc=False),
    )(_mf_body)
    return f(u_id.astype(jnp.int32), i_id.astype(jnp.int32),
             user_emb_w, user_bias_w.reshape(-1, 16),
             item_emb_w, item_bias_w.reshape(-1, 16), mean)


# speedup vs baseline: 2.0138x; 2.0138x over previous
import functools

import jax
import jax.numpy as jnp
from jax import lax
from jax.experimental import pallas as pl
from jax.experimental.pallas import tpu as pltpu
from jax.experimental.pallas import tpu_sc as plsc

B = 16384
EMB = 64
NC = 2
NS = 16
NW = NC * NS
BPW = B // NW
BLK = 4
NBLK = BPW // BLK


def _mf_body(u_id, i_id, uembT, ub1, iembT, ib1, mean, out,
             idx_u, idx_i, UW, IW, Ubw, Ibw, P, outv, meanv, sem):
    wid = lax.axis_index("s") * NC + lax.axis_index("c")
    base = wid * BPW

    pltpu.sync_copy(u_id.at[pl.ds(base, BPW)], idx_u)
    pltpu.sync_copy(i_id.at[pl.ds(base, BPW)], idx_i)
    pltpu.sync_copy(mean, meanv.at[pl.ds(0, 1)])

    m16 = meanv[...][0] * 0.0625
    iot = lax.iota(jnp.int32, 16)

    def outer(b, carry):
        uv = idx_u[pl.ds(b * 16, 16)]
        iv = idx_i[pl.ds(b * 16, 16)]
        for sb in range(4):
            cps = []
            for j in range(BLK):
                l = sb * BLK + j
                uid = uv[l]
                iid = iv[l]
                cu = pl.multiple_of(
                    lax.shift_right_logical(uid, 7) * 128, 128)
                ci = pl.multiple_of(
                    lax.shift_right_logical(iid, 7) * 128, 128)
                cps.append(pltpu.async_copy(
                    uembT.at[:, pl.ds(cu, 128)], UW.at[j], sem))
                cps.append(pltpu.async_copy(
                    iembT.at[:, pl.ds(ci, 128)], IW.at[j], sem))
                cps.append(pltpu.async_copy(
                    ub1.at[pl.ds(cu, 128)], Ubw.at[pl.ds(j * 128, 128)], sem))
                cps.append(pltpu.async_copy(
                    ib1.at[pl.ds(ci, 128)], Ibw.at[pl.ds(j * 128, 128)], sem))
            for cp in cps:
                cp.wait()
            for j in range(BLK):
                l = sb * BLK + j
                uid = uv[l]
                iid = iv[l]
                lu = lax.bitwise_and(uid, jnp.int32(127))
                li = lax.bitwise_and(iid, jnp.int32(127))
                luv = jnp.full((16,), 0, jnp.int32) + lu
                liv = jnp.full((16,), 0, jnp.int32) + li
                jv = jnp.full((16,), j, jnp.int32)
                acc = jnp.zeros((16,), jnp.float32)
                for gg in range(4):
                    fv = gg * 16 + iot
                    uu = plsc.load_gather(UW, [jv, fv, luv])
                    vv = plsc.load_gather(IW, [jv, fv, liv])
                    acc = acc + uu * vv
                ubv = plsc.load_gather(Ubw, [jnp.full((16,), j * 128, jnp.int32) + luv])
                ibv = plsc.load_gather(Ibw, [jnp.full((16,), j * 128, jnp.int32) + liv])
                acc = acc + (ubv + ibv) * 0.0625 + m16
                p = b * 16 + l
                P[pl.ds(p * 16, 16)] = acc
        return carry

    lax.fori_loop(0, BPW // 16, outer, 0)

    def phase2(b, carry):
        pvec = (b * 16 + iot) * 16
        acc = jnp.zeros((16,), jnp.float32)
        for l in range(16):
            acc = acc + plsc.load_gather(P, [pvec + jnp.full((16,), l, jnp.int32)])
        outv[pl.ds(b * 16, 16)] = acc
        return carry

    lax.fori_loop(0, BPW // 16, phase2, 0)

    pltpu.sync_copy(outv, out.at[pl.ds(base, BPW)])


@jax.jit
def kernel(u_id, i_id, user_emb_w, user_bias_w, item_emb_w, item_bias_w, mean):
    mesh = plsc.VectorSubcoreMesh(core_axis_name="c", subcore_axis_name="s")
    f = functools.partial(
        pl.kernel,
        out_type=jax.ShapeDtypeStruct((B,), jnp.float32),
        mesh=mesh,
        scratch_types=[
            pltpu.VMEM((BPW,), jnp.int32),
            pltpu.VMEM((BPW,), jnp.int32),
            pltpu.VMEM((BLK, EMB, 128), jnp.float32),
            pltpu.VMEM((BLK, EMB, 128), jnp.float32),
            pltpu.VMEM((BLK * 128,), jnp.float32),
            pltpu.VMEM((BLK * 128,), jnp.float32),
            pltpu.VMEM((BPW * 16,), jnp.float32),
            pltpu.VMEM((BPW,), jnp.float32),
            pltpu.VMEM((16,), jnp.float32),
            pltpu.SemaphoreType.DMA,
        ],
        compiler_params=pltpu.CompilerParams(
            needs_layout_passes=False, use_tc_tiling_on_sc=True),
    )(_mf_body)
    return f(u_id.astype(jnp.int32), i_id.astype(jnp.int32),
             user_emb_w.T, user_bias_w.reshape(-1),
             item_emb_w.T, item_bias_w.reshape(-1), mean)


# speedup vs baseline: 2.0426x; 1.0143x over previous
import functools

import jax
import jax.numpy as jnp
from jax import lax
from jax.experimental import pallas as pl
from jax.experimental.pallas import tpu as pltpu
from jax.experimental.pallas import tpu_sc as plsc

B = 16384
EMB = 64
NC = 2
NS = 16
NW = NC * NS
BPW = B // NW
NBODY = BPW // 4
IDXPAD = BPW + 32


def _mf_body(u_id, i_id, uembT, ub1, iembT, ib1, mean, out,
             idx_u, idx_i, UW_A, IW_A, UW_B, IW_B,
             Ub_A, Ib_A, Ub_B, Ib_B, P, outv, meanv, semA, semB):
    wid = lax.axis_index("s") * NC + lax.axis_index("c")
    base = wid * BPW

    pltpu.sync_copy(u_id.at[pl.ds(base, BPW)], idx_u.at[pl.ds(0, BPW)])
    pltpu.sync_copy(i_id.at[pl.ds(base, BPW)], idx_i.at[pl.ds(0, BPW)])
    pltpu.sync_copy(mean, meanv.at[pl.ds(0, 1)])

    m16 = meanv[...][0] * 0.0625
    iot = lax.iota(jnp.int32, 16)

    def enq(uv, iv, UWx, IWx, Ubx, Ibx, semx):
        for j in range(2):
            uid = uv[j]
            iid = iv[j]
            cu = pl.multiple_of(lax.shift_right_logical(uid, 7) * 128, 128)
            ci = pl.multiple_of(lax.shift_right_logical(iid, 7) * 128, 128)
            pltpu.async_copy(uembT.at[:, pl.ds(cu, 128)], UWx.at[j], semx)
            pltpu.async_copy(iembT.at[:, pl.ds(ci, 128)], IWx.at[j], semx)
            pltpu.async_copy(ub1.at[pl.ds(cu, 128)],
                             Ubx.at[pl.ds(j * 128, 128)], semx)
            pltpu.async_copy(ib1.at[pl.ds(ci, 128)],
                             Ibx.at[pl.ds(j * 128, 128)], semx)

    def drain(UWx, IWx, Ubx, Ibx, semx):
        for j in range(2):
            pltpu.make_async_copy(
                uembT.at[:, pl.ds(0, 128)], UWx.at[j], semx).wait()
            pltpu.make_async_copy(
                iembT.at[:, pl.ds(0, 128)], IWx.at[j], semx).wait()
            pltpu.make_async_copy(
                ub1.at[pl.ds(0, 128)], Ubx.at[pl.ds(j * 128, 128)], semx).wait()
            pltpu.make_async_copy(
                ib1.at[pl.ds(0, 128)], Ibx.at[pl.ds(j * 128, 128)], semx).wait()

    def comp(p0, uv, iv, UWx, IWx, Ubx, Ibx):
        for j in range(2):
            uid = uv[j]
            iid = iv[j]
            luv = jnp.full((16,), 0, jnp.int32) + lax.bitwise_and(uid, jnp.int32(127))
            liv = jnp.full((16,), 0, jnp.int32) + lax.bitwise_and(iid, jnp.int32(127))
            jv = jnp.full((16,), j, jnp.int32)
            acc = jnp.zeros((16,), jnp.float32)
            for g in range(4):
                fv = g * 16 + iot
                uu = plsc.load_gather(UWx, [jv, fv, luv])
                vv = plsc.load_gather(IWx, [jv, fv, liv])
                acc = acc + uu * vv
            ubv = plsc.load_gather(Ubx, [jnp.full((16,), j * 128, jnp.int32) + luv])
            ibv = plsc.load_gather(Ibx, [jnp.full((16,), j * 128, jnp.int32) + liv])
            acc = acc + (ubv + ibv) * 0.0625 + m16
            P[pl.ds((p0 + j) * 16, 16)] = acc

    enq(idx_u[pl.ds(0, 16)], idx_i[pl.ds(0, 16)],
        UW_A, IW_A, Ub_A, Ib_A, semA)
    enq(idx_u[pl.ds(2, 16)], idx_i[pl.ds(2, 16)],
        UW_B, IW_B, Ub_B, Ib_B, semB)

    def body(k, carry):
        pA = k * 4
        uvA = idx_u[pl.ds(pA, 16)]
        ivA = idx_i[pl.ds(pA, 16)]
        drain(UW_A, IW_A, Ub_A, Ib_A, semA)
        comp(pA, uvA, ivA, UW_A, IW_A, Ub_A, Ib_A)

        @pl.when(k < NBODY - 1)
        def _():
            enq(idx_u[pl.ds(pA + 4, 16)], idx_i[pl.ds(pA + 4, 16)],
                UW_A, IW_A, Ub_A, Ib_A, semA)

        pB = k * 4 + 2
        uvB = idx_u[pl.ds(pB, 16)]
        ivB = idx_i[pl.ds(pB, 16)]
        drain(UW_B, IW_B, Ub_B, Ib_B, semB)
        comp(pB, uvB, ivB, UW_B, IW_B, Ub_B, Ib_B)

        @pl.when(k < NBODY - 1)
        def _():
            enq(idx_u[pl.ds(pB + 4, 16)], idx_i[pl.ds(pB + 4, 16)],
                UW_B, IW_B, Ub_B, Ib_B, semB)

        return carry

    lax.fori_loop(0, NBODY, body, 0)

    def phase2(b, carry):
        pvec = (b * 16 + iot) * 16
        acc = jnp.zeros((16,), jnp.float32)
        for l in range(16):
            acc = acc + plsc.load_gather(P, [pvec + jnp.full((16,), l, jnp.int32)])
        outv[pl.ds(b * 16, 16)] = acc
        return carry

    lax.fori_loop(0, BPW // 16, phase2, 0)

    pltpu.sync_copy(outv, out.at[pl.ds(base, BPW)])


@jax.jit
def kernel(u_id, i_id, user_emb_w, user_bias_w, item_emb_w, item_bias_w, mean):
    mesh = plsc.VectorSubcoreMesh(core_axis_name="c", subcore_axis_name="s")
    f = functools.partial(
        pl.kernel,
        out_type=jax.ShapeDtypeStruct((B,), jnp.float32),
        mesh=mesh,
        scratch_types=[
            pltpu.VMEM((IDXPAD,), jnp.int32),
            pltpu.VMEM((IDXPAD,), jnp.int32),
            pltpu.VMEM((2, EMB, 128), jnp.float32),
            pltpu.VMEM((2, EMB, 128), jnp.float32),
            pltpu.VMEM((2, EMB, 128), jnp.float32),
            pltpu.VMEM((2, EMB, 128), jnp.float32),
            pltpu.VMEM((256,), jnp.float32),
            pltpu.VMEM((256,), jnp.float32),
            pltpu.VMEM((256,), jnp.float32),
            pltpu.VMEM((256,), jnp.float32),
            pltpu.VMEM((BPW * 16,), jnp.float32),
            pltpu.VMEM((BPW,), jnp.float32),
            pltpu.VMEM((16,), jnp.float32),
            pltpu.SemaphoreType.DMA,
            pltpu.SemaphoreType.DMA,
        ],
        compiler_params=pltpu.CompilerParams(
            needs_layout_passes=False, use_tc_tiling_on_sc=True),
    )(_mf_body)
    return f(u_id.astype(jnp.int32), i_id.astype(jnp.int32),
             user_emb_w.T, user_bias_w.reshape(-1),
             item_emb_w.T, item_bias_w.reshape(-1), mean)
